# baseline (speedup 1.0000x reference)
import jax
import jax.numpy as jnp
from jax import lax
from jax.experimental import pallas as pl
from jax.experimental.pallas import tpu as pltpu
from jax.experimental.pallas import tpu_sc as plsc

NC = 2
NS = 16
NW = NC * NS

B = 16384 * 50
D = 32
B_PER_W = B // NW
CHUNK = 800
N_CHUNKS = B_PER_W // CHUNK
NBUF = 4
DEPTH = NBUF - 1


def _gather_body(table_hbm, idx_hbm, out_hbm, idx_v, rows, gsems, wsems):
    wid = lax.axis_index("s") * NC + lax.axis_index("c")
    base = wid * B_PER_W

    pltpu.sync_copy(idx_hbm.at[pl.ds(base, B_PER_W)], idx_v)

    def gather(i):
        b = i % NBUF
        src = table_hbm.at[idx_v.at[pl.ds(i * CHUNK, CHUNK)]]
        return pltpu.async_copy(src, rows[b], gsems[b])

    def writeback(i):
        b = i % NBUF
        dst = out_hbm.at[pl.ds(base + i * CHUNK, CHUNK)]
        return pltpu.async_copy(rows[b], dst, wsems[b])

    g = {}
    w = {}
    for i in range(DEPTH):
        g[i] = gather(i)
    for i in range(N_CHUNKS):
        g[i].wait()
        w[i] = writeback(i)
        j = i + DEPTH
        if j < N_CHUNKS:
            if j >= NBUF:
                w[j - NBUF].wait()
            g[j] = gather(j)
    for i in range(max(0, N_CHUNKS - NBUF), N_CHUNKS):
        w[i].wait()


@jax.jit
def kernel(input, weight):
    idx = input.reshape(-1).astype(jnp.int32)
    mesh = plsc.VectorSubcoreMesh(core_axis_name="c", subcore_axis_name="s")
    out = pl.kernel(
        _gather_body,
        mesh=mesh,
        out_type=jax.ShapeDtypeStruct((B, D), jnp.float32),
        scratch_types=[
            pltpu.VMEM((B_PER_W,), jnp.int32),
            [pltpu.VMEM((CHUNK, D), jnp.float32) for _ in range(NBUF)],
            [pltpu.SemaphoreType.DMA for _ in range(NBUF)],
            [pltpu.SemaphoreType.DMA for _ in range(NBUF)],
        ],
        compiler_params=pltpu.CompilerParams(use_tc_tiling_on_sc=False),
    )(weight, idx)
    return out.reshape(input.shape + (D,))


# speedup vs baseline: 1.6158x; 1.6158x over previous
import jax
import jax.numpy as jnp
from jax import lax
from jax.experimental import pallas as pl
from jax.experimental.pallas import tpu as pltpu
from jax.experimental.pallas import tpu_sc as plsc

NC = 2
NS = 16
NW = NC * NS

BATCH = 16384
HIST = 50
B = BATCH * HIST
D = 32
B_PER_W = B // NW
ROWS_PER_W = BATCH // NW
CHUNK = 800
BR_PER_CHUNK = CHUNK // HIST
N_CHUNKS = B_PER_W // CHUNK


def _gather_body(table_hbm, idx_hbm, out_hbm, idx_v, rows0, rows1,
                 gsem0, gsem1, wsem):
    wid = lax.axis_index("s") * NC + lax.axis_index("c")
    base = wid * B_PER_W
    base_br = wid * ROWS_PER_W

    pltpu.sync_copy(idx_hbm.at[pl.ds(base, B_PER_W)], idx_v)

    rows = (rows0, rows1)
    gsems = (gsem0, gsem1)

    def gather(i, b):
        src = table_hbm.at[idx_v.at[pl.ds(i * CHUNK, CHUNK)]]
        return pltpu.async_copy(src, rows[b], gsems[b])

    def writeback(i, b):
        br0 = base_br + i * BR_PER_CHUNK
        return [
            pltpu.async_copy(
                rows[b].at[pl.ds(j * HIST, HIST)], out_hbm.at[br0 + j], wsem
            )
            for j in range(BR_PER_CHUNK)
        ]

    def body(g, carry):
        i0 = g * 2
        h0 = gather(i0, 0)
        h1 = gather(i0 + 1, 1)
        h0.wait()
        w0 = writeback(i0, 0)
        h1.wait()
        w1 = writeback(i0 + 1, 1)
        for h in w0 + w1:
            h.wait()
        return carry

    lax.fori_loop(0, N_CHUNKS // 2, body, 0)


@jax.jit
def kernel(input, weight):
    idx = input.reshape(-1).astype(jnp.int32)
    mesh = plsc.VectorSubcoreMesh(core_axis_name="c", subcore_axis_name="s")
    out = pl.kernel(
        _gather_body,
        mesh=mesh,
        out_type=jax.ShapeDtypeStruct((BATCH, HIST, D), jnp.float32),
        scratch_types=[
            pltpu.VMEM((B_PER_W,), jnp.int32),
            pltpu.VMEM((CHUNK, D), jnp.float32),
            pltpu.VMEM((CHUNK, D), jnp.float32),
            pltpu.SemaphoreType.DMA,
            pltpu.SemaphoreType.DMA,
            pltpu.SemaphoreType.DMA,
        ],
        compiler_params=pltpu.CompilerParams(use_tc_tiling_on_sc=False),
    )(weight, idx)
    return out
